# baseline (speedup 1.0000x reference)
import jax
import jax.numpy as jnp
from jax import lax
from jax.experimental import pallas as pl
from jax.experimental.pallas import tpu as pltpu
from jax.experimental.pallas import tpu_sc as plsc

_NW = 32


def _make_unpool(B, C, H, W):
    nplane = B * C
    nin = H * W
    Ho, Wo = 2 * H, 2 * W
    planes_per_w = nplane // _NW
    groups_per_row = W // 16
    zgroups_per_row = Wo // 16
    scat_iters = nin // 16

    def body(x_hbm, idx_hbm, out_hbm, x0, x1, i0, i1, out_v, sem_in, sem_out):
        wid = lax.axis_index("s") * 2 + lax.axis_index("c")
        base_plane = wid * planes_per_w
        zero16 = jnp.zeros((16,), jnp.float32)
        xbufs = [x0, x1]
        ibufs = [i0, i1]

        def start_in(plane, k):
            b = plane // C
            c = plane % C
            return [
                pltpu.async_copy(x_hbm.at[b, c], xbufs[k], sem_in),
                pltpu.async_copy(idx_hbm.at[b, c], ibufs[k], sem_in),
            ]

        def zero_fill():
            def zbody(r, c_):
                for u in range(zgroups_per_row):
                    out_v[r, pl.ds(u * 16, 16)] = zero16
                return c_

            lax.fori_loop(0, Ho, zbody, 0)

        def scatter(k, zeros):
            iv = ibufs[k]
            xv = xbufs[k]

            def sbody(j, c_):
                r = j // groups_per_row
                col = (j % groups_per_row) * 16
                idx = iv[r, pl.ds(col, 16)]
                vals = zero16 if zeros else xv[r, pl.ds(col, 16)]
                plsc.store_scatter(out_v, [idx // Wo, idx % Wo], vals)
                return c_

            lax.fori_loop(0, scat_iters, sbody, 0)

        cin = [None, None]
        cin[0] = start_in(base_plane, 0)
        zero_fill()
        pending = None
        for p in range(planes_per_w):
            k = p % 2
            cin[k][0].wait()
            cin[k][1].wait()
            if pending is not None:
                pending.wait()
                scatter(1 - k, zeros=True)
            if p + 1 < planes_per_w:
                cin[1 - k] = start_in(base_plane + p + 1, 1 - k)
            scatter(k, zeros=False)
            plane = base_plane + p
            pending = pltpu.async_copy(
                out_v, out_hbm.at[plane // C, plane % C], sem_out)
        pending.wait()

    mesh = plsc.VectorSubcoreMesh(core_axis_name="c", subcore_axis_name="s")
    return pl.kernel(
        body,
        mesh=mesh,
        compiler_params=pltpu.CompilerParams(needs_layout_passes=False),
        out_type=jax.ShapeDtypeStruct((B, C, Ho, Wo), jnp.float32),
        scratch_types=[
            pltpu.VMEM((H, W), jnp.float32),
            pltpu.VMEM((H, W), jnp.float32),
            pltpu.VMEM((H, W), jnp.int32),
            pltpu.VMEM((H, W), jnp.int32),
            pltpu.VMEM((Ho, Wo), jnp.float32),
            pltpu.SemaphoreType.DMA,
            pltpu.SemaphoreType.DMA,
        ],
    )


def kernel(x, indices):
    B, C, H, W = x.shape
    idx32 = indices.astype(jnp.int32)
    return _make_unpool(B, C, H, W)(x, idx32)


# speedup vs baseline: 3.1795x; 3.1795x over previous
import jax
import jax.numpy as jnp
from jax import lax
from jax.experimental import pallas as pl
from jax.experimental.pallas import tpu as pltpu
from jax.experimental.pallas import tpu_sc as plsc

_NW = 32


def _make_unpool(B, C, H, W):
    nplane = B * C
    nin = H * W
    Ho, Wo = 2 * H, 2 * W
    planes_per_w = nplane // _NW
    groups_per_row = W // 16
    scat_iters = nin // 16
    w_hi = Wo - 128
    assert Wo == 224

    def body(x_hbm, idx_hbm, out_hbm, x0, x1, i0, i1, outa, outb,
             sem_in, sem_out):
        wid = lax.axis_index("s") * 2 + lax.axis_index("c")
        base_plane = wid * planes_per_w
        zero16 = jnp.zeros((16,), jnp.float32)
        xbufs = [x0, x1]
        ibufs = [i0, i1]

        def start_in(plane, k):
            b = plane // C
            c = plane % C
            return [
                pltpu.async_copy(x_hbm.at[b, c], xbufs[k], sem_in),
                pltpu.async_copy(idx_hbm.at[b, c], ibufs[k], sem_in),
            ]

        def zero_fill():
            def zbody(r, c_):
                for u in range(8):
                    outa[r, pl.ds(u * 16, 16)] = zero16
                for u in range(6):
                    outb[r, pl.ds(u * 16, 16)] = zero16
                return c_

            lax.fori_loop(0, Ho, zbody, 0)

        def scatter(k, zeros):
            iv = ibufs[k]
            xv = xbufs[k]

            def sbody(j, c_):
                r = j // groups_per_row
                col = (j % groups_per_row) * 16
                idx = iv[r, pl.ds(col, 16)]
                vals = zero16 if zeros else xv[r, pl.ds(col, 16)]
                orow = jnp.right_shift((idx >> 5) * 9363, 16)
                ocol = idx - orow * 224
                in_a = ocol < 128
                plsc.store_scatter(outa, [orow, ocol], vals, mask=in_a)
                plsc.store_scatter(outb, [orow, ocol - 128], vals,
                                   mask=jnp.logical_not(in_a))
                return c_

            lax.fori_loop(0, scat_iters, sbody, 0)

        cin = [None, None]
        cin[0] = start_in(base_plane, 0)
        zero_fill()
        pending = None
        for p in range(planes_per_w):
            k = p % 2
            cin[k][0].wait()
            cin[k][1].wait()
            if pending is not None:
                pending[0].wait()
                pending[1].wait()
                scatter(1 - k, zeros=True)
            if p + 1 < planes_per_w:
                cin[1 - k] = start_in(base_plane + p + 1, 1 - k)
            scatter(k, zeros=False)
            plane = base_plane + p
            b = plane // C
            c = plane % C
            pending = [
                pltpu.async_copy(outa, out_hbm.at[b, c, :, pl.ds(0, 128)],
                                 sem_out),
                pltpu.async_copy(outb, out_hbm.at[b, c, :, pl.ds(128, w_hi)],
                                 sem_out),
            ]
        pending[0].wait()
        pending[1].wait()

    mesh = plsc.VectorSubcoreMesh(core_axis_name="c", subcore_axis_name="s")
    return pl.kernel(
        body,
        mesh=mesh,
        compiler_params=pltpu.CompilerParams(needs_layout_passes=False),
        out_type=jax.ShapeDtypeStruct((B, C, Ho, Wo), jnp.float32),
        scratch_types=[
            pltpu.VMEM((H, W), jnp.float32),
            pltpu.VMEM((H, W), jnp.float32),
            pltpu.VMEM((H, W), jnp.int32),
            pltpu.VMEM((H, W), jnp.int32),
            pltpu.VMEM((Ho, 128), jnp.float32),
            pltpu.VMEM((Ho, w_hi), jnp.float32),
            pltpu.SemaphoreType.DMA,
            pltpu.SemaphoreType.DMA,
        ],
    )


def kernel(x, indices):
    B, C, H, W = x.shape
    idx32 = indices.astype(jnp.int32)
    return _make_unpool(B, C, H, W)(x, idx32)


# speedup vs baseline: 4.2515x; 1.3371x over previous
import jax
import jax.numpy as jnp
from jax import lax
from jax.experimental import pallas as pl
from jax.experimental.pallas import tpu as pltpu
from jax.experimental.pallas import tpu_sc as plsc

_NW = 32


def _make_unpool(B, C, H, W):
    nplane = B * C
    Ho, Wo = 2 * H, 2 * W
    planes_per_w = nplane // _NW
    groups_per_row = W // 16
    w_hi = Wo - 128
    assert Wo == 224

    def body(x_hbm, idx_hbm, out_hbm, x0, x1, i0, i1, outa, outb,
             sem_in, sem_out):
        wid = lax.axis_index("s") * 2 + lax.axis_index("c")
        base_plane = wid * planes_per_w
        zero16 = jnp.zeros((16,), jnp.float32)
        xbufs = [x0, x1]
        ibufs = [i0, i1]

        def start_in(plane, k):
            b = plane // C
            c = plane % C
            return [
                pltpu.async_copy(x_hbm.at[b, c], xbufs[k], sem_in),
                pltpu.async_copy(idx_hbm.at[b, c], ibufs[k], sem_in),
            ]

        def zero_fill():
            def zbody(r, c_):
                for u in range(8):
                    outa[r, pl.ds(u * 16, 16)] = zero16
                for u in range(6):
                    outb[r, pl.ds(u * 16, 16)] = zero16
                return c_

            lax.fori_loop(0, Ho, zbody, 0)

        def scatter(k):
            iv = ibufs[k]
            xv = xbufs[k]

            def sbody(r, c_):
                for u in range(groups_per_row):
                    col = u * 16
                    idx = iv[r, pl.ds(col, 16)]
                    vals = xv[r, pl.ds(col, 16)]
                    orow = jnp.right_shift((idx >> 5) * 9363, 16)
                    ocol = idx - orow * 224
                    in_a = ocol < 128
                    plsc.store_scatter(outa, [orow, ocol], vals, mask=in_a)
                    plsc.store_scatter(outb, [orow, ocol - 128], vals,
                                       mask=jnp.logical_not(in_a))
                return c_

            lax.fori_loop(0, H, sbody, 0)

        cin = [None, None]
        cin[0] = start_in(base_plane, 0)
        zero_fill()
        pending = None
        for p in range(planes_per_w):
            k = p % 2
            cin[k][0].wait()
            cin[k][1].wait()
            if p + 1 < planes_per_w:
                cin[1 - k] = start_in(base_plane + p + 1, 1 - k)
            if pending is not None:
                pending[0].wait()
                pending[1].wait()
                zero_fill()
            scatter(k)
            plane = base_plane + p
            b = plane // C
            c = plane % C
            pending = [
                pltpu.async_copy(outa, out_hbm.at[b, c, :, pl.ds(0, 128)],
                                 sem_out),
                pltpu.async_copy(outb, out_hbm.at[b, c, :, pl.ds(128, w_hi)],
                                 sem_out),
            ]
        pending[0].wait()
        pending[1].wait()

    mesh = plsc.VectorSubcoreMesh(core_axis_name="c", subcore_axis_name="s")
    return pl.kernel(
        body,
        mesh=mesh,
        compiler_params=pltpu.CompilerParams(needs_layout_passes=False),
        out_type=jax.ShapeDtypeStruct((B, C, Ho, Wo), jnp.float32),
        scratch_types=[
            pltpu.VMEM((H, W), jnp.float32),
            pltpu.VMEM((H, W), jnp.float32),
            pltpu.VMEM((H, W), jnp.int32),
            pltpu.VMEM((H, W), jnp.int32),
            pltpu.VMEM((Ho, 128), jnp.float32),
            pltpu.VMEM((Ho, w_hi), jnp.float32),
            pltpu.SemaphoreType.DMA,
            pltpu.SemaphoreType.DMA,
        ],
    )


def kernel(x, indices):
    B, C, H, W = x.shape
    idx32 = indices.astype(jnp.int32)
    return _make_unpool(B, C, H, W)(x, idx32)


# speedup vs baseline: 4.2600x; 1.0020x over previous
import jax
import jax.numpy as jnp
from jax import lax
from jax.experimental import pallas as pl
from jax.experimental.pallas import tpu as pltpu
from jax.experimental.pallas import tpu_sc as plsc

_NW = 32


def _make_unpool(B, C, H, W):
    nplane = B * C
    Ho, Wo = 2 * H, 2 * W
    planes_per_w = nplane // _NW
    groups_per_row = W // 16
    w_hi = Wo - 128
    assert Wo == 224

    def body(x_hbm, idx_hbm, out_hbm, x0, x1, i0, i1, outa, outb,
             sem_in, sem_out):
        wid = lax.axis_index("s") * 2 + lax.axis_index("c")
        base_plane = wid * planes_per_w
        zero16 = jnp.zeros((16,), jnp.float32)
        xbufs = [x0, x1]
        ibufs = [i0, i1]

        def start_in(plane, k):
            b = plane // C
            c = plane % C
            return [
                pltpu.async_copy(x_hbm.at[b, c], xbufs[k], sem_in),
                pltpu.async_copy(idx_hbm.at[b, c], ibufs[k], sem_in),
            ]

        def zero_fill():
            def zbody(r, c_):
                for u in range(8):
                    outa[r, pl.ds(u * 16, 16)] = zero16
                for u in range(6):
                    outb[r, pl.ds(u * 16, 16)] = zero16
                return c_

            lax.fori_loop(0, Ho, zbody, 0)

        def scatter(k):
            iv = ibufs[k]
            xv = xbufs[k]

            def sbody(r, c_):
                for u in range(groups_per_row):
                    col = u * 16
                    idx = iv[r, pl.ds(col, 16)]
                    vals = xv[r, pl.ds(col, 16)]
                    orow = jnp.right_shift((idx >> 5) * 9363, 16)
                    ocol = idx - orow * 224
                    in_a = ocol < 128
                    plsc.store_scatter(outa, [orow, ocol], vals, mask=in_a)
                    plsc.store_scatter(outb, [orow, ocol - 128], vals,
                                       mask=jnp.logical_not(in_a))
                return c_

            lax.fori_loop(0, H, sbody, 0)

        cin = [None, None]
        cin[0] = start_in(base_plane, 0)
        zero_fill()
        pending = None
        for p in range(planes_per_w):
            k = p % 2
            if p + 1 < planes_per_w:
                cin[1 - k] = start_in(base_plane + p + 1, 1 - k)
            if pending is not None:
                pending[0].wait()
                pending[1].wait()
                zero_fill()
            cin[k][0].wait()
            cin[k][1].wait()
            scatter(k)
            plane = base_plane + p
            b = plane // C
            c = plane % C
            pending = [
                pltpu.async_copy(outa, out_hbm.at[b, c, :, pl.ds(0, 128)],
                                 sem_out),
                pltpu.async_copy(outb, out_hbm.at[b, c, :, pl.ds(128, w_hi)],
                                 sem_out),
            ]
        pending[0].wait()
        pending[1].wait()

    mesh = plsc.VectorSubcoreMesh(core_axis_name="c", subcore_axis_name="s")
    return pl.kernel(
        body,
        mesh=mesh,
        compiler_params=pltpu.CompilerParams(needs_layout_passes=False),
        out_type=jax.ShapeDtypeStruct((B, C, Ho, Wo), jnp.float32),
        scratch_types=[
            pltpu.VMEM((H, W), jnp.float32),
            pltpu.VMEM((H, W), jnp.float32),
            pltpu.VMEM((H, W), jnp.int32),
            pltpu.VMEM((H, W), jnp.int32),
            pltpu.VMEM((Ho, 128), jnp.float32),
            pltpu.VMEM((Ho, w_hi), jnp.float32),
            pltpu.SemaphoreType.DMA,
            pltpu.SemaphoreType.DMA,
        ],
    )


def kernel(x, indices):
    B, C, H, W = x.shape
    idx32 = indices.astype(jnp.int32)
    return _make_unpool(B, C, H, W)(x, idx32)


# speedup vs baseline: 4.4352x; 1.0411x over previous
import jax
import jax.numpy as jnp
from jax import lax
from jax.experimental import pallas as pl
from jax.experimental.pallas import tpu as pltpu
from jax.experimental.pallas import tpu_sc as plsc

_NW = 32


def _make_unpool(B, C, H, W):
    nplane = B * C
    Ho, Wo = 2 * H, 2 * W
    planes_per_w = nplane // _NW
    groups_per_row = W // 16
    w_hi = Wo - 128
    assert Wo == 224

    def body(x_hbm, idx_hbm, out_hbm, x0, x1, i_v, outa0, outa1, outb,
             sem_in, sem_a, sem_b):
        wid = lax.axis_index("s") * 2 + lax.axis_index("c")
        base_plane = wid * planes_per_w
        zero16 = jnp.zeros((16,), jnp.float32)
        xbufs = [x0, x1]
        oabufs = [outa0, outa1]

        def bc(plane):
            return plane // C, plane % C

        def start_x(plane, k):
            b, c = bc(plane)
            return pltpu.async_copy(x_hbm.at[b, c], xbufs[k], sem_in)

        def start_idx(plane):
            b, c = bc(plane)
            return pltpu.async_copy(idx_hbm.at[b, c], i_v, sem_in)

        def zero_a(oa):
            def zbody(r, c_):
                for u in range(8):
                    oa[r, pl.ds(u * 16, 16)] = zero16
                return c_

            lax.fori_loop(0, Ho, zbody, 0)

        def zero_b():
            def zbody(r, c_):
                for u in range(6):
                    outb[r, pl.ds(u * 16, 16)] = zero16
                return c_

            lax.fori_loop(0, Ho, zbody, 0)

        def scatter(k, oa):
            iv = i_v
            xv = xbufs[k]

            def sbody(r, c_):
                for u in range(groups_per_row):
                    col = u * 16
                    idx = iv[r, pl.ds(col, 16)]
                    vals = xv[r, pl.ds(col, 16)]
                    orow = jnp.right_shift((idx >> 5) * 9363, 16)
                    ocol = idx - orow * 224
                    in_a = ocol < 128
                    plsc.store_scatter(oa, [orow, ocol], vals, mask=in_a)
                    plsc.store_scatter(outb, [orow, ocol - 128], vals,
                                      mask=jnp.logical_not(in_a))
                return c_

            lax.fori_loop(0, H, sbody, 0)

        cx = [None, None]
        cx[0] = start_x(base_plane, 0)
        ci = start_idx(base_plane)
        zero_a(outa0)
        zero_a(outa1)
        zero_b()
        pend_a = [None, None]
        pend_b = None
        for p in range(planes_per_w):
            k = p % 2
            if p + 1 < planes_per_w:
                cx[1 - k] = start_x(base_plane + p + 1, 1 - k)
            oa = oabufs[k]
            if pend_a[k] is not None:
                pend_a[k].wait()
                zero_a(oa)
            if pend_b is not None:
                pend_b.wait()
                zero_b()
            cx[k].wait()
            ci.wait()
            scatter(k, oa)
            b, c = bc(base_plane + p)
            pend_a[k] = pltpu.async_copy(
                oa, out_hbm.at[b, c, :, pl.ds(0, 128)], sem_a)
            pend_b = pltpu.async_copy(
                outb, out_hbm.at[b, c, :, pl.ds(128, w_hi)], sem_b)
            if p + 1 < planes_per_w:
                ci = start_idx(base_plane + p + 1)
        pend_a[0].wait()
        pend_a[1].wait()
        pend_b.wait()

    mesh = plsc.VectorSubcoreMesh(core_axis_name="c", subcore_axis_name="s")
    return pl.kernel(
        body,
        mesh=mesh,
        compiler_params=pltpu.CompilerParams(needs_layout_passes=False),
        out_type=jax.ShapeDtypeStruct((B, C, Ho, Wo), jnp.float32),
        scratch_types=[
            pltpu.VMEM((H, W), jnp.float32),
            pltpu.VMEM((H, W), jnp.float32),
            pltpu.VMEM((H, W), jnp.int32),
            pltpu.VMEM((Ho, 128), jnp.float32),
            pltpu.VMEM((Ho, 128), jnp.float32),
            pltpu.VMEM((Ho, w_hi), jnp.float32),
            pltpu.SemaphoreType.DMA,
            pltpu.SemaphoreType.DMA,
            pltpu.SemaphoreType.DMA,
        ],
    )


def kernel(x, indices):
    B, C, H, W = x.shape
    idx32 = indices.astype(jnp.int32)
    return _make_unpool(B, C, H, W)(x, idx32)
